# Initial kernel scaffold; baseline (speedup 1.0000x reference)
import jax
import jax.numpy as jnp
from jax.experimental import pallas as pl


def kernel(x, edge_index, z, z_table, W1, b1, Wmu, bmu, Wls, bls):
    raise NotImplementedError("write your pallas kernel here")


# speedup vs baseline: 3.9895x; 3.9895x over previous
import functools

import jax
import jax.numpy as jnp
from jax import lax
from jax.experimental import pallas as pl
from jax.experimental.pallas import tpu as pltpu
from jax.experimental.pallas import tpu_sc as plsc

N = 10000
E = 320000
D_FEAT = 128
OUT = 64
D = 128
NTEC = 32
R = 320
NP = NTEC * R
CHUNK = 3200
BATCH = 128
CAP = E + CHUNK + 4800
BLK = 1024


def _sc_mesh():
    return plsc.VectorSubcoreMesh(core_axis_name="c", subcore_axis_name="s")


def _wid():
    return lax.axis_index("s") * 2 + lax.axis_index("c")


def _filter_body(ei, esrc, edst, cnt, deg,
                 schunk, dchunk, sstage, dstage, degv, cntv):
    wid = _wid()
    lo = wid * R
    hi = jnp.minimum(lo + R, N)
    zf = jnp.zeros((16,), jnp.float32)

    for i in range(R // 16 + 1):
        degv[pl.ds(16 * i, 16)] = zf

    def vbody(k, ccnt):
        s16 = schunk[pl.ds(k * 16, 16)]
        d16 = dchunk[pl.ds(k * 16, 16)]
        m = (d16 >= lo) & (d16 < hi)
        mi = m.astype(jnp.int32)
        incl = plsc.cumsum(mi)
        pos = ccnt + incl - mi
        plsc.store_scatter(sstage, [pos], s16, mask=m)
        plsc.store_scatter(dstage, [pos], d16 - lo, mask=m)
        return ccnt + jnp.sum(mi)

    e0 = jnp.where(lax.iota(jnp.int32, 16) == 0, 1.0, 0.0)

    def dbody(i, _):
        dv = dstage[pl.ds(i, 16)][0]
        plsc.addupdate(degv.at[pl.ds(dv, 16)], e0)
        return 0

    def chunk_body(ci, total):
        pltpu.sync_copy(ei.at[pl.ds(pl.multiple_of(ci * CHUNK, 8), CHUNK)], schunk)
        pltpu.sync_copy(ei.at[pl.ds(pl.multiple_of(E + ci * CHUNK, 8), CHUNK)], dchunk)
        ccnt = lax.fori_loop(0, CHUNK // 16, vbody, 0)
        sstage[pl.ds(ccnt, 16)] = jnp.zeros((16,), jnp.int32)
        dstage[pl.ds(ccnt, 16)] = jnp.full((16,), R, jnp.int32)
        lax.fori_loop(0, ccnt, dbody, 0)
        pltpu.sync_copy(sstage, esrc.at[pl.ds(pl.multiple_of(wid * CAP + total, 8), CHUNK + 16)])
        pltpu.sync_copy(dstage, edst.at[pl.ds(pl.multiple_of(wid * CAP + total, 8), CHUNK + 16)])
        return total + ((ccnt + 15) // 16) * 16

    total = lax.fori_loop(0, E // CHUNK, chunk_body, 0)

    for j in range(BATCH // 16):
        sstage[pl.ds(j * 16, 16)] = jnp.zeros((16,), jnp.int32)
        dstage[pl.ds(j * 16, 16)] = jnp.full((16,), R, jnp.int32)
    pltpu.sync_copy(sstage.at[pl.ds(0, BATCH)], esrc.at[pl.ds(pl.multiple_of(wid * CAP + total, 8), BATCH)])
    pltpu.sync_copy(dstage.at[pl.ds(0, BATCH)], edst.at[pl.ds(pl.multiple_of(wid * CAP + total, 8), BATCH)])
    npad = ((total + BATCH - 1) // BATCH) * BATCH
    cntv[...] = jnp.broadcast_to(npad, (16,)).astype(jnp.int32)
    pltpu.sync_copy(cntv, cnt.at[pl.ds(pl.multiple_of(wid * 16, 8), 16)])
    pltpu.sync_copy(degv.at[pl.ds(0, R)], deg.at[pl.ds(pl.multiple_of(wid * R, 8), R)])


@functools.partial(
    pl.kernel,
    out_type=(
        jax.ShapeDtypeStruct((NTEC * CAP,), jnp.int32),
        jax.ShapeDtypeStruct((NTEC * CAP,), jnp.int32),
        jax.ShapeDtypeStruct((NTEC * 16,), jnp.int32),
        jax.ShapeDtypeStruct((NP,), jnp.float32),
    ),
    mesh=_sc_mesh(),
    compiler_params=pltpu.CompilerParams(needs_layout_passes=False),
    scratch_types=[
        pltpu.VMEM((CHUNK,), jnp.int32),
        pltpu.VMEM((CHUNK,), jnp.int32),
        pltpu.VMEM((CHUNK + 16,), jnp.int32),
        pltpu.VMEM((CHUNK + 16,), jnp.int32),
        pltpu.VMEM((R + 16,), jnp.float32),
        pltpu.VMEM((16,), jnp.int32),
    ],
)
def _sc_filter(ei, esrc, edst, cnt, deg, *scratch):
    _filter_body(ei, esrc, edst, cnt, deg, *scratch)


def _conv_body(y, esrc, edst, cnt, out, sidx, dloc, rows, acc, cntv, gsem):
    wid = _wid()
    zf = jnp.zeros((16,), jnp.float32)

    def zrow(r, _):
        for c in range(8):
            acc[r, pl.ds(c * 16, 16)] = zf
        return 0

    lax.fori_loop(0, R + 1, zrow, 0)

    pltpu.sync_copy(cnt.at[pl.ds(pl.multiple_of(wid * 16, 8), 16)], cntv)
    n = cntv[pl.ds(0, 16)][0]

    def edge(e, _):
        dl = dloc[pl.ds(e, 16)][0]
        for c in range(8):
            v = rows[e, pl.ds(c * 16, 16)]
            plsc.addupdate(acc.at[dl, pl.ds(c * 16, 16)], v)
        return 0

    def batch(g, _):
        off = g * BATCH
        pltpu.sync_copy(esrc.at[pl.ds(pl.multiple_of(wid * CAP + off, 8), BATCH)], sidx)
        pltpu.sync_copy(edst.at[pl.ds(pl.multiple_of(wid * CAP + off, 8), BATCH)], dloc.at[pl.ds(0, BATCH)])
        pltpu.async_copy(y.at[sidx], rows, gsem).wait()
        lax.fori_loop(0, BATCH, edge, 0)
        return 0

    lax.fori_loop(0, n // BATCH, batch, 0)
    pltpu.sync_copy(acc.at[pl.ds(0, R)], out.at[pl.ds(pl.multiple_of(wid * R, 8), R)])


@functools.partial(
    pl.kernel,
    out_type=jax.ShapeDtypeStruct((NP, D), jnp.float32),
    mesh=_sc_mesh(),
    compiler_params=pltpu.CompilerParams(needs_layout_passes=False),
    scratch_types=[
        pltpu.VMEM((BATCH,), jnp.int32),
        pltpu.VMEM((BATCH + 16,), jnp.int32),
        pltpu.VMEM((BATCH, D), jnp.float32),
        pltpu.VMEM((R + 1, D), jnp.float32),
        pltpu.VMEM((16,), jnp.int32),
        pltpu.SemaphoreType.DMA,
    ],
)
def _sc_conv(y, esrc, edst, cnt, out, *scratch):
    _conv_body(y, esrc, edst, cnt, out, *scratch)


def _tc_a_body(x_ref, z_ref, zt_ref, w1_ref, deg_ref, y1_ref, dinv_ref):
    w1 = w1_ref[...]
    t2 = jnp.dot(zt_ref[...], w1[D_FEAT:, :], preferred_element_type=jnp.float32)
    xw = jnp.dot(x_ref[...], w1[:D_FEAT, :], preferred_element_type=jnp.float32)
    zrow = jnp.where(z_ref[...] == 1, t2[1:2, :], t2[0:1, :])
    di = lax.rsqrt(deg_ref[...] + 1.0)
    y1_ref[...] = (xw + zrow) * di
    dinv_ref[...] = di


def _tc_b_body(acc1_ref, y1_ref, dinv_ref, b1_ref, wcat_ref, y2_ref):
    di = dinv_ref[...]
    h = jnp.maximum(di * (acc1_ref[...] + y1_ref[...]) + b1_ref[...], 0.0)
    y2_ref[...] = jnp.dot(h, wcat_ref[...], preferred_element_type=jnp.float32) * di


def _tc_c_body(acc2_ref, y2_ref, dinv_ref, bcat_ref, out_ref):
    di = dinv_ref[...]
    out_ref[...] = di * (acc2_ref[...] + y2_ref[...]) + bcat_ref[...]


def _row_spec(width):
    return pl.BlockSpec((BLK, width), lambda i: (i, 0))


def _full_spec(shape):
    return pl.BlockSpec(shape, lambda i: tuple(0 for _ in shape))


_tc_a = pl.pallas_call(
    _tc_a_body,
    grid=(NP // BLK,),
    in_specs=[
        _row_spec(D_FEAT),
        _row_spec(1),
        _full_spec((2, OUT)),
        _full_spec((D_FEAT + OUT, 2 * OUT)),
        _row_spec(1),
    ],
    out_specs=[_row_spec(D), _row_spec(1)],
    out_shape=[
        jax.ShapeDtypeStruct((NP, D), jnp.float32),
        jax.ShapeDtypeStruct((NP, 1), jnp.float32),
    ],
)

_tc_b = pl.pallas_call(
    _tc_b_body,
    grid=(NP // BLK,),
    in_specs=[
        _row_spec(D),
        _row_spec(D),
        _row_spec(1),
        _full_spec((1, 2 * OUT)),
        _full_spec((2 * OUT, 2 * OUT)),
    ],
    out_specs=_row_spec(D),
    out_shape=jax.ShapeDtypeStruct((NP, D), jnp.float32),
)

_tc_c = pl.pallas_call(
    _tc_c_body,
    grid=(NP // BLK,),
    in_specs=[
        _row_spec(D),
        _row_spec(D),
        _row_spec(1),
        _full_spec((1, 2 * OUT)),
    ],
    out_specs=_row_spec(D),
    out_shape=jax.ShapeDtypeStruct((NP, D), jnp.float32),
)


def kernel(x, edge_index, z, z_table, W1, b1, Wmu, bmu, Wls, bls):
    xp = jnp.pad(x, ((0, NP - N), (0, 0)))
    zp = jnp.pad(z, (0, NP - N)).reshape(NP, 1)
    wcat = jnp.concatenate([Wmu, Wls], axis=1)
    bcat = jnp.concatenate([bmu, bls]).reshape(1, 2 * OUT)
    b1_2d = b1.reshape(1, 2 * OUT)

    esrc, edst, cnt, deg = _sc_filter(edge_index.reshape(2 * E))
    y1, dinv = _tc_a(xp, zp, z_table, W1, deg.reshape(NP, 1))
    acc1 = _sc_conv(y1, esrc, edst, cnt)
    y2 = _tc_b(acc1, y1, dinv, b1_2d, wcat)
    acc2 = _sc_conv(y2, esrc, edst, cnt)
    outc = _tc_c(acc2, y2, dinv, bcat)
    mu = outc[:N, :OUT]
    logstd = outc[:N, OUT:]
    return (mu, logstd)


# speedup vs baseline: 15.7192x; 3.9402x over previous
import functools

import jax
import jax.numpy as jnp
from jax import lax
from jax.experimental import pallas as pl
from jax.experimental.pallas import tpu as pltpu
from jax.experimental.pallas import tpu_sc as plsc

N = 10000
E = 320000
D_FEAT = 128
OUT = 64
D = 128
NSC = 2
NSUB = 16
NTEC = NSC * NSUB
EPT = E // NTEC
BATCH = 128
NB = EPT // BATCH
TAIL = EPT - NB * BATCH
NP = 10240
RPS = NP // NSUB
BLK = 1024


def _sc_mesh():
    return plsc.VectorSubcoreMesh(core_axis_name="c", subcore_axis_name="s")


_sc_params = pltpu.CompilerParams()


def _deg_body(ei, out, dall, degv, tmp, acc, part_sh):
    cid = lax.axis_index("c")
    sid = lax.axis_index("s")
    w = sid * NSC + cid
    zf = jnp.zeros((16,), jnp.float32)
    e0 = jnp.where(lax.iota(jnp.int32, 16) == 0, 1.0, 0.0)

    def zr(i, _):
        degv[pl.ds(i * 16, 16)] = zf
        return 0

    lax.fori_loop(0, NP // 16, zr, 0)
    off = pl.multiple_of(w * EPT, 8)
    pltpu.sync_copy(ei.at[pl.ds(E + off, EPT)], dall.at[pl.ds(0, EPT)])

    def ebody(e, _):
        dv = dall[pl.ds(e, 16)][0]
        plsc.addupdate(degv.at[pl.ds(dv, 16)], e0)
        return 0

    lax.fori_loop(0, EPT, ebody, 0)
    pltpu.sync_copy(degv, part_sh.at[sid])
    plsc.subcore_barrier()
    base = pl.multiple_of(sid * RPS, 8)
    for c in range(RPS // 16):
        acc[pl.ds(c * 16, 16)] = zf
    for j in range(NSUB):
        pltpu.sync_copy(part_sh.at[j, pl.ds(base, RPS)], tmp)
        for c in range(RPS // 16):
            v = tmp[pl.ds(c * 16, 16)]
            plsc.addupdate(acc.at[pl.ds(c * 16, 16)], v)
    pltpu.sync_copy(acc, out.at[cid, pl.ds(base, RPS)])


@functools.partial(
    pl.kernel,
    out_type=jax.ShapeDtypeStruct((NSC, NP), jnp.float32),
    mesh=_sc_mesh(),
    compiler_params=_sc_params,
    scratch_types=[
        pltpu.VMEM((EPT + 16,), jnp.int32),
        pltpu.VMEM((NP,), jnp.float32),
        pltpu.VMEM((RPS,), jnp.float32),
        pltpu.VMEM((RPS,), jnp.float32),
        pltpu.VMEM_SHARED((NSUB, NP), jnp.float32),
    ],
)
def _sc_deg(ei, out, *scratch):
    _deg_body(ei, out, *scratch)


def _conv_body(y, ei, out, sidx, didx, rows, tidx, tdidx, trows, zbuf, acc_sh, gsem):
    cid = lax.axis_index("c")
    sid = lax.axis_index("s")
    w = sid * NSC + cid
    zf = jnp.zeros((16,), jnp.float32)

    def zr(r, _):
        for c in range(D // 16):
            zbuf[r, pl.ds(c * 16, 16)] = zf
        return 0

    lax.fori_loop(0, BATCH, zr, 0)
    for r in range(RPS // BATCH):
        pltpu.sync_copy(
            zbuf, acc_sh.at[pl.ds(pl.multiple_of(sid * RPS + r * BATCH, 8), BATCH)])
    plsc.subcore_barrier()

    def batch(g, _):
        off = pl.multiple_of(w * EPT + g * BATCH, 8)
        pltpu.sync_copy(ei.at[pl.ds(off, BATCH)], sidx)
        pltpu.sync_copy(ei.at[pl.ds(E + off, BATCH)], didx)
        pltpu.async_copy(y.at[sidx], rows, gsem).wait()
        pltpu.sync_copy(rows, acc_sh.at[didx], add=True)
        return 0

    lax.fori_loop(0, NB, batch, 0)
    toff = pl.multiple_of(w * EPT + NB * BATCH, 8)
    pltpu.sync_copy(ei.at[pl.ds(toff, TAIL)], tidx)
    pltpu.sync_copy(ei.at[pl.ds(E + toff, TAIL)], tdidx)
    pltpu.async_copy(y.at[tidx], trows, gsem).wait()
    pltpu.sync_copy(trows, acc_sh.at[tdidx], add=True)
    plsc.subcore_barrier()
    pltpu.sync_copy(acc_sh.at[pl.ds(pl.multiple_of(sid * RPS, 8), RPS)],
                    out.at[cid, pl.ds(pl.multiple_of(sid * RPS, 8), RPS)])


@functools.partial(
    pl.kernel,
    out_type=jax.ShapeDtypeStruct((NSC, NP, D), jnp.float32),
    mesh=_sc_mesh(),
    compiler_params=_sc_params,
    scratch_types=[
        pltpu.VMEM((BATCH,), jnp.int32),
        pltpu.VMEM((BATCH,), jnp.int32),
        pltpu.VMEM((BATCH, D), jnp.float32),
        pltpu.VMEM((TAIL,), jnp.int32),
        pltpu.VMEM((TAIL,), jnp.int32),
        pltpu.VMEM((TAIL, D), jnp.float32),
        pltpu.VMEM((BATCH, D), jnp.float32),
        pltpu.VMEM_SHARED((NP, D), jnp.float32),
        pltpu.SemaphoreType.DMA,
    ],
)
def _sc_conv(y, ei, out, *scratch):
    _conv_body(y, ei, out, *scratch)


def _tc_a_body(x_ref, z_ref, zt_ref, w1_ref, dega_ref, degb_ref, y1_ref, dinv_ref):
    w1 = w1_ref[...]
    t2 = jnp.dot(zt_ref[...], w1[D_FEAT:, :], preferred_element_type=jnp.float32)
    xw = jnp.dot(x_ref[...], w1[:D_FEAT, :], preferred_element_type=jnp.float32)
    zrow = jnp.where(z_ref[...] == 1, t2[1:2, :], t2[0:1, :])
    di = lax.rsqrt(dega_ref[...] + degb_ref[...] + 1.0)
    y1_ref[...] = (xw + zrow) * di
    dinv_ref[...] = di


def _tc_b_body(acca_ref, accb_ref, y1_ref, dinv_ref, b1_ref, wcat_ref, y2_ref):
    di = dinv_ref[...]
    h = jnp.maximum(
        di * (acca_ref[...] + accb_ref[...] + y1_ref[...]) + b1_ref[...], 0.0)
    y2_ref[...] = jnp.dot(h, wcat_ref[...], preferred_element_type=jnp.float32) * di


def _tc_c_body(acca_ref, accb_ref, y2_ref, dinv_ref, bcat_ref, out_ref):
    di = dinv_ref[...]
    out_ref[...] = di * (acca_ref[...] + accb_ref[...] + y2_ref[...]) + bcat_ref[...]


def _row_spec(width):
    return pl.BlockSpec((BLK, width), lambda i: (i, 0))


def _prow_spec(width, core):
    return pl.BlockSpec((1, BLK, width), lambda i, c=core: (c, i, 0))


def _full_spec(shape):
    return pl.BlockSpec(shape, lambda i: tuple(0 for _ in shape))


_tc_a = pl.pallas_call(
    _tc_a_body,
    grid=(NP // BLK,),
    in_specs=[
        _row_spec(D_FEAT),
        _row_spec(1),
        _full_spec((2, OUT)),
        _full_spec((D_FEAT + OUT, 2 * OUT)),
        _row_spec(1),
        _row_spec(1),
    ],
    out_specs=[_row_spec(D), _row_spec(1)],
    out_shape=[
        jax.ShapeDtypeStruct((NP, D), jnp.float32),
        jax.ShapeDtypeStruct((NP, 1), jnp.float32),
    ],
)

_tc_b = pl.pallas_call(
    _tc_b_body,
    grid=(NP // BLK,),
    in_specs=[
        _row_spec(D),
        _row_spec(D),
        _row_spec(D),
        _row_spec(1),
        _full_spec((1, 2 * OUT)),
        _full_spec((2 * OUT, 2 * OUT)),
    ],
    out_specs=_row_spec(D),
    out_shape=jax.ShapeDtypeStruct((NP, D), jnp.float32),
)

_tc_c = pl.pallas_call(
    _tc_c_body,
    grid=(NP // BLK,),
    in_specs=[
        _row_spec(D),
        _row_spec(D),
        _row_spec(D),
        _row_spec(1),
        _full_spec((1, 2 * OUT)),
    ],
    out_specs=_row_spec(D),
    out_shape=jax.ShapeDtypeStruct((NP, D), jnp.float32),
)


def kernel(x, edge_index, z, z_table, W1, b1, Wmu, bmu, Wls, bls):
    xp = jnp.pad(x, ((0, NP - N), (0, 0)))
    zp = jnp.pad(z, (0, NP - N)).reshape(NP, 1)
    wcat = jnp.concatenate([Wmu, Wls], axis=1)
    bcat = jnp.concatenate([bmu, bls]).reshape(1, 2 * OUT)
    b1_2d = b1.reshape(1, 2 * OUT)
    ei = edge_index.reshape(2 * E)

    deg2 = _sc_deg(ei)
    y1, dinv = _tc_a(xp, zp, z_table, W1, deg2[0].reshape(NP, 1), deg2[1].reshape(NP, 1))
    acc1 = _sc_conv(y1, ei)
    y2 = _tc_b(acc1[0], acc1[1], y1, dinv, b1_2d, wcat)
    acc2 = _sc_conv(y2, ei)
    outc = _tc_c(acc2[0], acc2[1], y2, dinv, bcat)
    mu = outc[:N, :OUT]
    logstd = outc[:N, OUT:]
    return (mu, logstd)


# speedup vs baseline: 22.9814x; 1.4620x over previous
import functools

import jax
import jax.numpy as jnp
from jax import lax
from jax.experimental import pallas as pl
from jax.experimental.pallas import tpu as pltpu
from jax.experimental.pallas import tpu_sc as plsc

N = 10000
E = 320000
D_FEAT = 128
OUT = 64
D = 128
NSC = 2
NSUB = 16
NTEC = NSC * NSUB
EPT = E // NTEC
BATCH = 128
NB = EPT // BATCH
TAIL = EPT - NB * BATCH
NP = 10240
RPS = NP // NSUB
BLK = 1024


def _sc_mesh():
    return plsc.VectorSubcoreMesh(core_axis_name="c", subcore_axis_name="s")


_sc_params = pltpu.CompilerParams()


def _deg_body(ei, out, dall, degv, tmp, acc, part_sh):
    cid = lax.axis_index("c")
    sid = lax.axis_index("s")
    w = sid * NSC + cid
    zf = jnp.zeros((16,), jnp.float32)
    e0 = jnp.where(lax.iota(jnp.int32, 16) == 0, 1.0, 0.0)

    def zr(i, _):
        degv[pl.ds(i * 16, 16)] = zf
        return 0

    lax.fori_loop(0, NP // 16, zr, 0)
    off = pl.multiple_of(w * EPT, 8)
    pltpu.sync_copy(ei.at[pl.ds(E + off, EPT)], dall.at[pl.ds(0, EPT)])

    def ebody(i, _):
        for u in range(4):
            dv = dall[pl.ds(i * 4 + u, 16)][0]
            plsc.addupdate(degv.at[pl.ds(dv, 16)], e0)
        return 0

    lax.fori_loop(0, EPT // 4, ebody, 0)
    pltpu.sync_copy(degv, part_sh.at[sid])
    plsc.subcore_barrier()
    base = pl.multiple_of(sid * RPS, 8)
    for c in range(RPS // 16):
        acc[pl.ds(c * 16, 16)] = zf
    for j in range(NSUB):
        pltpu.sync_copy(part_sh.at[j, pl.ds(base, RPS)], tmp)
        for c in range(RPS // 16):
            v = tmp[pl.ds(c * 16, 16)]
            plsc.addupdate(acc.at[pl.ds(c * 16, 16)], v)
    pltpu.sync_copy(acc, out.at[cid, pl.ds(base, RPS)])


@functools.partial(
    pl.kernel,
    out_type=jax.ShapeDtypeStruct((NSC, NP), jnp.float32),
    mesh=_sc_mesh(),
    compiler_params=_sc_params,
    scratch_types=[
        pltpu.VMEM((EPT + 16,), jnp.int32),
        pltpu.VMEM((NP,), jnp.float32),
        pltpu.VMEM((RPS,), jnp.float32),
        pltpu.VMEM((RPS,), jnp.float32),
        pltpu.VMEM_SHARED((NSUB, NP), jnp.float32),
    ],
)
def _sc_deg(ei, out, *scratch):
    _deg_body(ei, out, *scratch)


def _conv_body(y, ei, out, sidx0, sidx1, didx0, didx1, tidx, tdidx, trows,
               rows, acc_sh, isa, isb, gsa, gsb):
    cid = lax.axis_index("c")
    sid = lax.axis_index("s")
    w = sid * NSC + cid
    zf = jnp.zeros((16,), jnp.float32)

    sidx = (sidx0, sidx1)
    didx = (didx0, didx1)
    isem = (isa, isb)
    gsem = (gsa, gsb)

    def issue_idx(b, g):
        off = pl.multiple_of(w * EPT + g * BATCH, 8)
        pltpu.async_copy(ei.at[pl.ds(off, BATCH)], sidx[b], isem[b])
        pltpu.async_copy(ei.at[pl.ds(E + off, BATCH)], didx[b], isem[b])

    def wait_idx(b, g):
        off = pl.multiple_of(w * EPT + g * BATCH, 8)
        pltpu.make_async_copy(ei.at[pl.ds(off, BATCH)], sidx[b], isem[b]).wait()
        pltpu.make_async_copy(ei.at[pl.ds(E + off, BATCH)], didx[b], isem[b]).wait()

    def issue_gather(b):
        pltpu.async_copy(y.at[sidx[b]], rows.at[b], gsem[b])

    def wait_gather(b):
        pltpu.make_async_copy(y.at[sidx[b]], rows.at[b], gsem[b]).wait()

    def zr(r, _):
        for c in range(D // 16):
            rows[0, r, pl.ds(c * 16, 16)] = zf
        return 0

    lax.fori_loop(0, BATCH, zr, 0)
    issue_idx(0, 0)
    for r in range(RPS // BATCH):
        pltpu.sync_copy(
            rows.at[0],
            acc_sh.at[pl.ds(pl.multiple_of(sid * RPS + r * BATCH, 8), BATCH)])
    plsc.subcore_barrier()
    wait_idx(0, 0)
    issue_gather(0)
    issue_idx(1, 1)

    def pair(i, _):
        for b in (0, 1):
            g = i * 2 + b
            nxt = g + 1
            wait_gather(b)

            @pl.when(nxt < NB)
            def _():
                wait_idx(1 - b, nxt)
                issue_gather(1 - b)

            pltpu.sync_copy(rows.at[b], acc_sh.at[didx[b]], add=True)

            @pl.when(nxt + 1 < NB)
            def _():
                issue_idx(b, nxt + 1)
        return 0

    lax.fori_loop(0, NB // 2, pair, 0)

    toff = pl.multiple_of(w * EPT + NB * BATCH, 8)
    pltpu.sync_copy(ei.at[pl.ds(toff, TAIL)], tidx)
    pltpu.sync_copy(ei.at[pl.ds(E + toff, TAIL)], tdidx)
    pltpu.async_copy(y.at[tidx], trows, gsa).wait()
    pltpu.sync_copy(trows, acc_sh.at[tdidx], add=True)
    plsc.subcore_barrier()
    pltpu.sync_copy(acc_sh.at[pl.ds(pl.multiple_of(sid * RPS, 8), RPS)],
                    out.at[cid, pl.ds(pl.multiple_of(sid * RPS, 8), RPS)])


@functools.partial(
    pl.kernel,
    out_type=jax.ShapeDtypeStruct((NSC, NP, D), jnp.float32),
    mesh=_sc_mesh(),
    compiler_params=_sc_params,
    scratch_types=[
        pltpu.VMEM((BATCH,), jnp.int32),
        pltpu.VMEM((BATCH,), jnp.int32),
        pltpu.VMEM((BATCH,), jnp.int32),
        pltpu.VMEM((BATCH,), jnp.int32),
        pltpu.VMEM((TAIL,), jnp.int32),
        pltpu.VMEM((TAIL,), jnp.int32),
        pltpu.VMEM((TAIL, D), jnp.float32),
        pltpu.VMEM((2, BATCH, D), jnp.float32),
        pltpu.VMEM_SHARED((NP, D), jnp.float32),
        pltpu.SemaphoreType.DMA,
        pltpu.SemaphoreType.DMA,
        pltpu.SemaphoreType.DMA,
        pltpu.SemaphoreType.DMA,
    ],
)
def _sc_conv(y, ei, out, *scratch):
    _conv_body(y, ei, out, *scratch)


def _tc_a_body(x_ref, z_ref, zt_ref, w1_ref, dega_ref, degb_ref, y1_ref, dinv_ref):
    w1 = w1_ref[...]
    t2 = jnp.dot(zt_ref[...], w1[D_FEAT:, :], preferred_element_type=jnp.float32)
    xw = jnp.dot(x_ref[...], w1[:D_FEAT, :], preferred_element_type=jnp.float32)
    zrow = jnp.where(z_ref[...] == 1, t2[1:2, :], t2[0:1, :])
    di = lax.rsqrt(dega_ref[...] + degb_ref[...] + 1.0)
    y1_ref[...] = (xw + zrow) * di
    dinv_ref[...] = di


def _tc_b_body(acca_ref, accb_ref, y1_ref, dinv_ref, b1_ref, wcat_ref, y2_ref):
    di = dinv_ref[...]
    h = jnp.maximum(
        di * (acca_ref[...] + accb_ref[...] + y1_ref[...]) + b1_ref[...], 0.0)
    y2_ref[...] = jnp.dot(h, wcat_ref[...], preferred_element_type=jnp.float32) * di


def _tc_c_body(acca_ref, accb_ref, y2_ref, dinv_ref, bcat_ref, out_ref):
    di = dinv_ref[...]
    out_ref[...] = di * (acca_ref[...] + accb_ref[...] + y2_ref[...]) + bcat_ref[...]


def _row_spec(width):
    return pl.BlockSpec((BLK, width), lambda i: (i, 0))


def _prow_spec(width, core):
    return pl.BlockSpec((1, BLK, width), lambda i, c=core: (c, i, 0))


def _full_spec(shape):
    return pl.BlockSpec(shape, lambda i: tuple(0 for _ in shape))


_tc_a = pl.pallas_call(
    _tc_a_body,
    grid=(NP // BLK,),
    in_specs=[
        _row_spec(D_FEAT),
        _row_spec(1),
        _full_spec((2, OUT)),
        _full_spec((D_FEAT + OUT, 2 * OUT)),
        _row_spec(1),
        _row_spec(1),
    ],
    out_specs=[_row_spec(D), _row_spec(1)],
    out_shape=[
        jax.ShapeDtypeStruct((NP, D), jnp.float32),
        jax.ShapeDtypeStruct((NP, 1), jnp.float32),
    ],
)

_tc_b = pl.pallas_call(
    _tc_b_body,
    grid=(NP // BLK,),
    in_specs=[
        _row_spec(D),
        _row_spec(D),
        _row_spec(D),
        _row_spec(1),
        _full_spec((1, 2 * OUT)),
        _full_spec((2 * OUT, 2 * OUT)),
    ],
    out_specs=_row_spec(D),
    out_shape=jax.ShapeDtypeStruct((NP, D), jnp.float32),
)

_tc_c = pl.pallas_call(
    _tc_c_body,
    grid=(NP // BLK,),
    in_specs=[
        _row_spec(D),
        _row_spec(D),
        _row_spec(D),
        _row_spec(1),
        _full_spec((1, 2 * OUT)),
    ],
    out_specs=_row_spec(D),
    out_shape=jax.ShapeDtypeStruct((NP, D), jnp.float32),
)


def kernel(x, edge_index, z, z_table, W1, b1, Wmu, bmu, Wls, bls):
    xp = jnp.pad(x, ((0, NP - N), (0, 0)))
    zp = jnp.pad(z, (0, NP - N)).reshape(NP, 1)
    wcat = jnp.concatenate([Wmu, Wls], axis=1)
    bcat = jnp.concatenate([bmu, bls]).reshape(1, 2 * OUT)
    b1_2d = b1.reshape(1, 2 * OUT)
    ei = edge_index.reshape(2 * E)

    deg2 = _sc_deg(ei)
    y1, dinv = _tc_a(xp, zp, z_table, W1, deg2[0].reshape(NP, 1), deg2[1].reshape(NP, 1))
    acc1 = _sc_conv(y1, ei)
    y2 = _tc_b(acc1[0], acc1[1], y1, dinv, b1_2d, wcat)
    acc2 = _sc_conv(y2, ei)
    outc = _tc_c(acc2[0], acc2[1], y2, dinv, bcat)
    mu = outc[:N, :OUT]
    logstd = outc[:N, OUT:]
    return (mu, logstd)


# speedup vs baseline: 25.3259x; 1.1020x over previous
import functools

import jax
import jax.numpy as jnp
from jax import lax
from jax.experimental import pallas as pl
from jax.experimental.pallas import tpu as pltpu
from jax.experimental.pallas import tpu_sc as plsc

N = 10000
E = 320000
D_FEAT = 128
OUT = 64
D = 128
NSC = 2
NSUB = 16
NTEC = NSC * NSUB
EPT = E // NTEC
BATCH = 128
NB = EPT // BATCH
TAIL = EPT - NB * BATCH
NP = 10240
RPS = NP // NSUB
BLK = 1024


def _sc_mesh():
    return plsc.VectorSubcoreMesh(core_axis_name="c", subcore_axis_name="s")


_sc_params = pltpu.CompilerParams()


HALF = NP // 2


def _deg_body(ei, out, dall, hist, degv, tmp, acc, part_sh):
    cid = lax.axis_index("c")
    sid = lax.axis_index("s")
    w = sid * NSC + cid
    zf = jnp.zeros((16,), jnp.float32)
    ones = zf + 1.0
    lane = lax.iota(jnp.int32, 16)

    off = pl.multiple_of(w * EPT, 8)
    pltpu.sync_copy(ei.at[pl.ds(E + off, EPT)], dall)

    for h in range(2):
        lo = h * HALF

        def zh(i, _):
            hist[pl.ds(i * 16, 16)] = zf
            return 0

        lax.fori_loop(0, HALF, zh, 0)

        def ebody(k, _):
            dv = dall[pl.ds(k * 16, 16)]
            m = (dv >= lo) & (dv < lo + HALF)
            idx = jnp.where(m, (dv - lo) * 16 + lane, lane)
            plsc.addupdate_scatter(hist, [idx], ones, mask=m)
            return 0

        lax.fori_loop(0, EPT // 16, ebody, 0)

        def rbody(g, _):
            base = g * 256
            t = zf
            for j in range(16):
                t = t + plsc.load_gather(hist, [base + lane * 16 + j])
            degv[pl.ds(lo + g * 16, 16)] = t
            return 0

        lax.fori_loop(0, HALF // 16, rbody, 0)

    pltpu.sync_copy(degv, part_sh.at[sid])
    plsc.subcore_barrier()
    base = pl.multiple_of(sid * RPS, 8)
    for c in range(RPS // 16):
        acc[pl.ds(c * 16, 16)] = zf
    for j in range(NSUB):
        pltpu.sync_copy(part_sh.at[j, pl.ds(base, RPS)], tmp)
        for c in range(RPS // 16):
            v = tmp[pl.ds(c * 16, 16)]
            plsc.addupdate(acc.at[pl.ds(c * 16, 16)], v)
    pltpu.sync_copy(acc, out.at[cid, pl.ds(base, RPS)])


@functools.partial(
    pl.kernel,
    out_type=jax.ShapeDtypeStruct((NSC, NP), jnp.float32),
    mesh=_sc_mesh(),
    compiler_params=pltpu.CompilerParams(needs_layout_passes=False),
    scratch_types=[
        pltpu.VMEM((EPT,), jnp.int32),
        pltpu.VMEM((HALF * 16,), jnp.float32),
        pltpu.VMEM((NP,), jnp.float32),
        pltpu.VMEM((RPS,), jnp.float32),
        pltpu.VMEM((RPS,), jnp.float32),
        pltpu.VMEM_SHARED((NSUB, NP), jnp.float32),
    ],
)
def _sc_deg(ei, out, *scratch):
    _deg_body(ei, out, *scratch)


def _conv_body(y, ei, out, sidx0, sidx1, didx0, didx1, tidx, tdidx, trows,
               rows, acc_sh, isa, isb, gsa, gsb):
    cid = lax.axis_index("c")
    sid = lax.axis_index("s")
    w = sid * NSC + cid
    zf = jnp.zeros((16,), jnp.float32)

    sidx = (sidx0, sidx1)
    didx = (didx0, didx1)
    isem = (isa, isb)
    gsem = (gsa, gsb)

    def issue_idx(b, g):
        off = pl.multiple_of(w * EPT + g * BATCH, 8)
        pltpu.async_copy(ei.at[pl.ds(off, BATCH)], sidx[b], isem[b])
        pltpu.async_copy(ei.at[pl.ds(E + off, BATCH)], didx[b], isem[b])

    def wait_idx(b, g):
        off = pl.multiple_of(w * EPT + g * BATCH, 8)
        pltpu.make_async_copy(ei.at[pl.ds(off, BATCH)], sidx[b], isem[b]).wait()
        pltpu.make_async_copy(ei.at[pl.ds(E + off, BATCH)], didx[b], isem[b]).wait()

    def issue_gather(b):
        pltpu.async_copy(y.at[sidx[b]], rows.at[b], gsem[b])

    def wait_gather(b):
        pltpu.make_async_copy(y.at[sidx[b]], rows.at[b], gsem[b]).wait()

    def zr(r, _):
        for c in range(D // 16):
            rows[0, r, pl.ds(c * 16, 16)] = zf
        return 0

    lax.fori_loop(0, BATCH, zr, 0)
    issue_idx(0, 0)
    for r in range(RPS // BATCH):
        pltpu.sync_copy(
            rows.at[0],
            acc_sh.at[pl.ds(pl.multiple_of(sid * RPS + r * BATCH, 8), BATCH)])
    plsc.subcore_barrier()
    wait_idx(0, 0)
    issue_gather(0)
    issue_idx(1, 1)

    def pair(i, _):
        for b in (0, 1):
            g = i * 2 + b
            nxt = g + 1
            wait_gather(b)

            @pl.when(nxt < NB)
            def _():
                wait_idx(1 - b, nxt)
                issue_gather(1 - b)

            pltpu.sync_copy(rows.at[b], acc_sh.at[didx[b]], add=True)

            @pl.when(nxt + 1 < NB)
            def _():
                issue_idx(b, nxt + 1)
        return 0

    lax.fori_loop(0, NB // 2, pair, 0)

    toff = pl.multiple_of(w * EPT + NB * BATCH, 8)
    pltpu.sync_copy(ei.at[pl.ds(toff, TAIL)], tidx)
    pltpu.sync_copy(ei.at[pl.ds(E + toff, TAIL)], tdidx)
    pltpu.async_copy(y.at[tidx], trows, gsa).wait()
    pltpu.sync_copy(trows, acc_sh.at[tdidx], add=True)
    plsc.subcore_barrier()
    pltpu.sync_copy(acc_sh.at[pl.ds(pl.multiple_of(sid * RPS, 8), RPS)],
                    out.at[cid, pl.ds(pl.multiple_of(sid * RPS, 8), RPS)])


@functools.partial(
    pl.kernel,
    out_type=jax.ShapeDtypeStruct((NSC, NP, D), jnp.float32),
    mesh=_sc_mesh(),
    compiler_params=_sc_params,
    scratch_types=[
        pltpu.VMEM((BATCH,), jnp.int32),
        pltpu.VMEM((BATCH,), jnp.int32),
        pltpu.VMEM((BATCH,), jnp.int32),
        pltpu.VMEM((BATCH,), jnp.int32),
        pltpu.VMEM((TAIL,), jnp.int32),
        pltpu.VMEM((TAIL,), jnp.int32),
        pltpu.VMEM((TAIL, D), jnp.float32),
        pltpu.VMEM((2, BATCH, D), jnp.float32),
        pltpu.VMEM_SHARED((NP, D), jnp.float32),
        pltpu.SemaphoreType.DMA,
        pltpu.SemaphoreType.DMA,
        pltpu.SemaphoreType.DMA,
        pltpu.SemaphoreType.DMA,
    ],
)
def _sc_conv(y, ei, out, *scratch):
    _conv_body(y, ei, out, *scratch)


def _tc_a_body(x_ref, z_ref, zt_ref, w1_ref, dega_ref, degb_ref, y1_ref, dinv_ref):
    w1 = w1_ref[...]
    t2 = jnp.dot(zt_ref[...], w1[D_FEAT:, :], preferred_element_type=jnp.float32)
    xw = jnp.dot(x_ref[...], w1[:D_FEAT, :], preferred_element_type=jnp.float32)
    zrow = jnp.where(z_ref[...] == 1, t2[1:2, :], t2[0:1, :])
    di = lax.rsqrt(dega_ref[...] + degb_ref[...] + 1.0)
    y1_ref[...] = (xw + zrow) * di
    dinv_ref[...] = di


def _tc_b_body(acca_ref, accb_ref, y1_ref, dinv_ref, b1_ref, wcat_ref, y2_ref):
    di = dinv_ref[...]
    h = jnp.maximum(
        di * (acca_ref[...] + accb_ref[...] + y1_ref[...]) + b1_ref[...], 0.0)
    y2_ref[...] = jnp.dot(h, wcat_ref[...], preferred_element_type=jnp.float32) * di


def _tc_c_body(acca_ref, accb_ref, y2_ref, dinv_ref, bcat_ref, out_ref):
    di = dinv_ref[...]
    out_ref[...] = di * (acca_ref[...] + accb_ref[...] + y2_ref[...]) + bcat_ref[...]


def _row_spec(width):
    return pl.BlockSpec((BLK, width), lambda i: (i, 0))


def _prow_spec(width, core):
    return pl.BlockSpec((1, BLK, width), lambda i, c=core: (c, i, 0))


def _full_spec(shape):
    return pl.BlockSpec(shape, lambda i: tuple(0 for _ in shape))


_tc_a = pl.pallas_call(
    _tc_a_body,
    grid=(NP // BLK,),
    in_specs=[
        _row_spec(D_FEAT),
        _row_spec(1),
        _full_spec((2, OUT)),
        _full_spec((D_FEAT + OUT, 2 * OUT)),
        _row_spec(1),
        _row_spec(1),
    ],
    out_specs=[_row_spec(D), _row_spec(1)],
    out_shape=[
        jax.ShapeDtypeStruct((NP, D), jnp.float32),
        jax.ShapeDtypeStruct((NP, 1), jnp.float32),
    ],
)

_tc_b = pl.pallas_call(
    _tc_b_body,
    grid=(NP // BLK,),
    in_specs=[
        _row_spec(D),
        _row_spec(D),
        _row_spec(D),
        _row_spec(1),
        _full_spec((1, 2 * OUT)),
        _full_spec((2 * OUT, 2 * OUT)),
    ],
    out_specs=_row_spec(D),
    out_shape=jax.ShapeDtypeStruct((NP, D), jnp.float32),
)

_tc_c = pl.pallas_call(
    _tc_c_body,
    grid=(NP // BLK,),
    in_specs=[
        _row_spec(D),
        _row_spec(D),
        _row_spec(D),
        _row_spec(1),
        _full_spec((1, 2 * OUT)),
    ],
    out_specs=_row_spec(D),
    out_shape=jax.ShapeDtypeStruct((NP, D), jnp.float32),
)


def kernel(x, edge_index, z, z_table, W1, b1, Wmu, bmu, Wls, bls):
    xp = jnp.pad(x, ((0, NP - N), (0, 0)))
    zp = jnp.pad(z, (0, NP - N)).reshape(NP, 1)
    wcat = jnp.concatenate([Wmu, Wls], axis=1)
    bcat = jnp.concatenate([bmu, bls]).reshape(1, 2 * OUT)
    b1_2d = b1.reshape(1, 2 * OUT)
    ei = edge_index.reshape(2 * E)

    deg2 = _sc_deg(ei)
    y1, dinv = _tc_a(xp, zp, z_table, W1, deg2[0].reshape(NP, 1), deg2[1].reshape(NP, 1))
    acc1 = _sc_conv(y1, ei)
    y2 = _tc_b(acc1[0], acc1[1], y1, dinv, b1_2d, wcat)
    acc2 = _sc_conv(y2, ei)
    outc = _tc_c(acc2[0], acc2[1], y2, dinv, bcat)
    mu = outc[:N, :OUT]
    logstd = outc[:N, OUT:]
    return (mu, logstd)
